# Initial kernel scaffold; baseline (speedup 1.0000x reference)
import jax
import jax.numpy as jnp
from jax.experimental import pallas as pl


def kernel(features_s, features_t, labels):
    raise NotImplementedError("write your pallas kernel here")


# speedup vs baseline: 2.1010x; 2.1010x over previous
import functools

import jax
import jax.numpy as jnp
from jax.experimental import pallas as pl
from jax.experimental.pallas import tpu as pltpu

_NCLS = 19
_PAD = 32
_TEMP = 0.1


def _body(fs_ref, ft_ref, lab_ref, out_ref, acc_s, acc_t, cnt, *, nsteps, rows, ratio):
    i = pl.program_id(0)

    @pl.when(i == 0)
    def _init():
        acc_s[...] = jnp.zeros_like(acc_s)
        acc_t[...] = jnp.zeros_like(acc_t)
        cnt[...] = jnp.zeros_like(cnt)

    x_s = fs_ref[0]
    x_t = ft_ref[0]
    lab = lab_ref[0].astype(jnp.float32)

    lw = lab.shape[1]
    w = lw // ratio
    jc = jax.lax.broadcasted_iota(jnp.int32, (lw, w), 0)
    wc = jax.lax.broadcasted_iota(jnp.int32, (lw, w), 1)
    sel_col = (jc == ratio * wc).astype(jnp.float32)
    lab_c = jax.lax.dot(lab, sel_col)
    rr = jax.lax.broadcasted_iota(jnp.int32, (rows, ratio * rows), 0)
    jr = jax.lax.broadcasted_iota(jnp.int32, (rows, ratio * rows), 1)
    sel_row = (jr == ratio * rr).astype(jnp.float32)
    lab_ds = jax.lax.dot(sel_row, lab_c)

    kio = jax.lax.broadcasted_iota(jnp.int32, (rows, w, _PAD), 2).astype(jnp.float32)
    oh = (lab_ds[:, :, None] == kio).astype(jnp.float32)
    oh2 = oh.reshape(rows * w, _PAD)

    hp = jax.lax.Precision.HIGHEST
    acc_s[...] += jax.lax.dot(x_s, oh2, precision=hp)
    acc_t[...] += jax.lax.dot(x_t, oh2, precision=hp)
    cnt[...] += jnp.sum(oh2, axis=0, keepdims=True)

    @pl.when(i == nsteps - 1)
    def _fin():
        counts = cnt[0]
        present = counts > 0.0
        denom = jnp.where(present, counts, 1.0)
        mean_s = acc_s[...] / denom[None, :]
        mean_t = acc_t[...] / denom[None, :]
        ns = jnp.sqrt(jnp.sum(mean_s * mean_s, axis=0, keepdims=True))
        nt = jnp.sqrt(jnp.sum(mean_t * mean_t, axis=0, keepdims=True))
        s_n = jnp.where(present[None, :], mean_s / jnp.maximum(ns, 1e-12), 0.0)
        t_n = jnp.where(present[None, :], mean_t / jnp.maximum(nt, 1e-12), 0.0)
        logits = (
            jax.lax.dot_general(s_n, t_n, (((0,), (0,)), ((), ())), precision=hp)
            / _TEMP
        )
        ri = jax.lax.broadcasted_iota(jnp.int32, (_PAD, _PAD), 0)
        ci = jax.lax.broadcasted_iota(jnp.int32, (_PAD, _PAD), 1)
        e = jnp.where(ci < _NCLS, jnp.exp(logits), 0.0)
        row_sum = jnp.sum(e, axis=1)
        diag = jnp.sum(jnp.where(ri == ci, logits, 0.0), axis=1)
        per_cls = jnp.log(row_sum) - diag
        n_present = jnp.maximum(jnp.sum(jnp.where(present, 1.0, 0.0)), 1.0)
        loss = jnp.sum(jnp.where(present, per_cls, 0.0)) / n_present
        out_ref[...] = jnp.broadcast_to(loss, (1, 1))


@jax.jit
def kernel(features_s, features_t, labels):
    B, C, H, W = features_s.shape
    Lh, Lw = labels.shape[1], labels.shape[2]
    ratio = Lh // H
    fs = features_s.reshape(B, C, H * W)
    ft = features_t.reshape(B, C, H * W)
    rows = 8
    np_blk = rows * W
    steps_per_b = H // rows
    nsteps = B * steps_per_b

    out = pl.pallas_call(
        functools.partial(_body, nsteps=nsteps, rows=rows, ratio=ratio),
        grid=(nsteps,),
        in_specs=[
            pl.BlockSpec((1, C, np_blk), lambda i: (i // steps_per_b, 0, i % steps_per_b)),
            pl.BlockSpec((1, C, np_blk), lambda i: (i // steps_per_b, 0, i % steps_per_b)),
            pl.BlockSpec(
                (1, ratio * rows, Lw), lambda i: (i // steps_per_b, i % steps_per_b, 0)
            ),
        ],
        out_specs=pl.BlockSpec((1, 1), lambda i: (0, 0)),
        out_shape=jax.ShapeDtypeStruct((1, 1), jnp.float32),
        scratch_shapes=[
            pltpu.VMEM((C, _PAD), jnp.float32),
            pltpu.VMEM((C, _PAD), jnp.float32),
            pltpu.VMEM((1, _PAD), jnp.float32),
        ],
    )(fs, ft, labels)
    return out[0, 0]


# speedup vs baseline: 2.9292x; 1.3942x over previous
import functools

import jax
import jax.numpy as jnp
from jax.experimental import pallas as pl
from jax.experimental.pallas import tpu as pltpu

_NCLS = 19
_PAD = 32
_TEMP = 0.1


def _body(fs_ref, ft_ref, lab_ref, out_ref, acc_s, acc_t, cnt, *, nsteps, rows, ratio):
    i = pl.program_id(0)

    @pl.when(i == 0)
    def _init():
        acc_s[...] = jnp.zeros_like(acc_s)
        acc_t[...] = jnp.zeros_like(acc_t)
        cnt[...] = jnp.zeros_like(cnt)

    x_s = fs_ref[0]
    x_t = ft_ref[0]
    lab = lab_ref[0].astype(jnp.float32)

    lw = lab.shape[1]
    w = lw // ratio
    jc = jax.lax.broadcasted_iota(jnp.int32, (lw, w), 0)
    wc = jax.lax.broadcasted_iota(jnp.int32, (lw, w), 1)
    sel_col = (jc == ratio * wc).astype(jnp.float32)
    lab_c = jax.lax.dot(lab, sel_col)
    rr = jax.lax.broadcasted_iota(jnp.int32, (rows, ratio * rows), 0)
    jr = jax.lax.broadcasted_iota(jnp.int32, (rows, ratio * rows), 1)
    sel_row = (jr == ratio * rr).astype(jnp.float32)
    lab_ds = jax.lax.dot(sel_row, lab_c)

    kio = jax.lax.broadcasted_iota(jnp.int32, (rows, w, _PAD), 2).astype(jnp.float32)
    oh = (lab_ds[:, :, None] == kio).astype(jnp.float32)
    oh2 = oh.reshape(rows * w, _PAD).astype(jnp.bfloat16)

    f32 = jnp.float32
    s_hi = x_s.astype(jnp.bfloat16)
    s_lo = (x_s - s_hi.astype(f32)).astype(jnp.bfloat16)
    t_hi = x_t.astype(jnp.bfloat16)
    t_lo = (x_t - t_hi.astype(f32)).astype(jnp.bfloat16)
    acc_s[...] += jax.lax.dot(s_hi, oh2, preferred_element_type=f32) + jax.lax.dot(
        s_lo, oh2, preferred_element_type=f32
    )
    acc_t[...] += jax.lax.dot(t_hi, oh2, preferred_element_type=f32) + jax.lax.dot(
        t_lo, oh2, preferred_element_type=f32
    )
    cnt[...] += jnp.sum(oh, axis=0)

    @pl.when(i == nsteps - 1)
    def _fin():
        hp = jax.lax.Precision.HIGHEST
        counts = jnp.sum(cnt[...], axis=0)
        present = counts > 0.0
        denom = jnp.where(present, counts, 1.0)
        mean_s = acc_s[...] / denom[None, :]
        mean_t = acc_t[...] / denom[None, :]
        ns = jnp.sqrt(jnp.sum(mean_s * mean_s, axis=0, keepdims=True))
        nt = jnp.sqrt(jnp.sum(mean_t * mean_t, axis=0, keepdims=True))
        s_n = jnp.where(present[None, :], mean_s / jnp.maximum(ns, 1e-12), 0.0)
        t_n = jnp.where(present[None, :], mean_t / jnp.maximum(nt, 1e-12), 0.0)
        logits = (
            jax.lax.dot_general(s_n, t_n, (((0,), (0,)), ((), ())), precision=hp)
            / _TEMP
        )
        ri = jax.lax.broadcasted_iota(jnp.int32, (_PAD, _PAD), 0)
        ci = jax.lax.broadcasted_iota(jnp.int32, (_PAD, _PAD), 1)
        e = jnp.where(ci < _NCLS, jnp.exp(logits), 0.0)
        row_sum = jnp.sum(e, axis=1)
        diag = jnp.sum(jnp.where(ri == ci, logits, 0.0), axis=1)
        per_cls = jnp.log(row_sum) - diag
        n_present = jnp.maximum(jnp.sum(jnp.where(present, 1.0, 0.0)), 1.0)
        loss = jnp.sum(jnp.where(present, per_cls, 0.0)) / n_present
        out_ref[...] = jnp.broadcast_to(loss, (1, 1))


@jax.jit
def kernel(features_s, features_t, labels):
    B, C, H, W = features_s.shape
    Lh, Lw = labels.shape[1], labels.shape[2]
    ratio = Lh // H
    fs = features_s.reshape(B, C, H * W)
    ft = features_t.reshape(B, C, H * W)
    rows = 16
    np_blk = rows * W
    steps_per_b = H // rows
    nsteps = B * steps_per_b

    out = pl.pallas_call(
        functools.partial(_body, nsteps=nsteps, rows=rows, ratio=ratio),
        grid=(nsteps,),
        in_specs=[
            pl.BlockSpec((1, C, np_blk), lambda i: (i // steps_per_b, 0, i % steps_per_b)),
            pl.BlockSpec((1, C, np_blk), lambda i: (i // steps_per_b, 0, i % steps_per_b)),
            pl.BlockSpec(
                (1, ratio * rows, Lw), lambda i: (i // steps_per_b, i % steps_per_b, 0)
            ),
        ],
        out_specs=pl.BlockSpec((1, 1), lambda i: (0, 0)),
        out_shape=jax.ShapeDtypeStruct((1, 1), jnp.float32),
        scratch_shapes=[
            pltpu.VMEM((C, _PAD), jnp.float32),
            pltpu.VMEM((C, _PAD), jnp.float32),
            pltpu.VMEM((W, _PAD), jnp.float32),
        ],
    )(fs, ft, labels)
    return out[0, 0]
